# baseline (speedup 1.0000x reference)
import jax
import jax.numpy as jnp
from jax import lax
from jax.experimental import pallas as pl
from jax.experimental.pallas import tpu as pltpu
from jax.experimental.pallas import tpu_sc as plsc

_NUM_WORKERS = 32
_NBUF = 8
_LAG = 4
_LANES = 128
_GIDX = 56


def kernel(x, table):
    batch, seq = x.shape
    _, emb = table.shape

    xp = jnp.pad(x, ((0, 0), (0, _LANES - seq)))

    rows_per_worker = batch // _NUM_WORKERS
    num_groups = rows_per_worker // _NBUF
    mesh = plsc.VectorSubcoreMesh(core_axis_name="c", subcore_axis_name="s")

    @pl.kernel(
        out_type=jax.ShapeDtypeStruct((batch, seq, emb), table.dtype),
        mesh=mesh,
        compiler_params=pltpu.CompilerParams(use_tc_tiling_on_sc=False),
        scratch_types=[
            pltpu.VMEM((rows_per_worker, _LANES), jnp.int32),
            [pltpu.VMEM((_GIDX, emb), table.dtype) for _ in range(_NBUF)],
            [pltpu.SemaphoreType.DMA for _ in range(_NBUF)],
            [pltpu.SemaphoreType.DMA for _ in range(_NBUF)],
        ],
    )
    def gather_kernel(table_hbm, xp_hbm, out_hbm, idx_all, rows, gsem, wsem):
        wid = lax.axis_index("s") * 2 + lax.axis_index("c")
        rbase = wid * rows_per_worker
        pltpu.sync_copy(xp_hbm.at[pl.ds(rbase, rows_per_worker)], idx_all)

        def start_gather(r, b):
            pltpu.async_copy(
                table_hbm.at[idx_all.at[r, pl.ds(0, _GIDX)]], rows[b], gsem[b]
            )

        def wait_gather(r, b):
            pltpu.make_async_copy(
                table_hbm.at[idx_all.at[r, pl.ds(0, _GIDX)]], rows[b], gsem[b]
            ).wait()

        def start_wb(r, b):
            pltpu.async_copy(
                rows[b].at[pl.ds(0, seq)], out_hbm.at[rbase + r], wsem[b]
            )

        def wait_wb(r, b):
            pltpu.make_async_copy(
                rows[b].at[pl.ds(0, seq)], out_hbm.at[rbase + r], wsem[b]
            ).wait()

        for i in range(_NBUF):
            start_gather(i, i)
            if i >= _LAG:
                d = i - _LAG
                wait_gather(d, d % _NBUF)
                start_wb(d, d % _NBUF)

        @pl.loop(1, num_groups)
        def _(k):
            r0 = k * _NBUF
            for i in range(_NBUF):
                r = r0 + i
                wait_wb(r - _NBUF, i)
                start_gather(r, i)
                d = r - _LAG
                bd = (i + _NBUF - _LAG) % _NBUF
                wait_gather(d, bd)
                start_wb(d, bd)

        for d in range(rows_per_worker - _LAG, rows_per_worker):
            wait_gather(d, d % _NBUF)
            start_wb(d, d % _NBUF)
        for b in range(_NBUF):
            wait_wb(rows_per_worker - _NBUF + b, b)

    return gather_kernel(table, xp)


# speedup vs baseline: 2.7262x; 2.7262x over previous
import jax
import jax.numpy as jnp
from jax import lax
from jax.experimental import pallas as pl
from jax.experimental.pallas import tpu as pltpu
from jax.experimental.pallas import tpu_sc as plsc

_NUM_WORKERS = 32
_CHUNK = 128
_NBUF = 4
_LAG = 2
_LANES = 128
_VL = 16


def kernel(x, table):
    batch, seq = x.shape
    _, emb = table.shape
    n = batch * seq

    def pad_body(x_ref, o_ref):
        v = x_ref[...]
        z = jnp.zeros((v.shape[0], _LANES - seq), v.dtype)
        o_ref[...] = jnp.concatenate([v, z], axis=1)

    xpad = pl.pallas_call(
        pad_body,
        out_shape=jax.ShapeDtypeStruct((batch, _LANES), x.dtype),
    )(x)

    rows_per_worker = batch // _NUM_WORKERS
    per_worker = rows_per_worker * seq
    num_chunks = per_worker // _CHUNK
    num_groups = num_chunks // _NBUF
    mesh = plsc.VectorSubcoreMesh(core_axis_name="c", subcore_axis_name="s")

    @pl.kernel(
        out_type=jax.ShapeDtypeStruct((n, emb), table.dtype),
        mesh=mesh,
        compiler_params=pltpu.CompilerParams(
            use_tc_tiling_on_sc=False, needs_layout_passes=False
        ),
        scratch_types=[
            pltpu.VMEM((rows_per_worker, _LANES), jnp.int32),
            pltpu.VMEM((per_worker,), jnp.int32),
            [pltpu.VMEM((_CHUNK, emb), table.dtype) for _ in range(_NBUF)],
            [pltpu.SemaphoreType.DMA for _ in range(_NBUF)],
            [pltpu.SemaphoreType.DMA for _ in range(_NBUF)],
        ],
    )
    def gather_kernel(table_hbm, xp_hbm, out_hbm, slab, idx_flat, rows,
                      gsem, wsem):
        wid = lax.axis_index("s") * 2 + lax.axis_index("c")
        rbase = wid * rows_per_worker
        base = wid * per_worker
        pltpu.sync_copy(xp_hbm.at[pl.ds(rbase, rows_per_worker)], slab)

        lane = lax.iota(jnp.int32, _VL)
        nsub = (seq + _VL - 1) // _VL

        @pl.loop(0, rows_per_worker)
        def _(r):
            dbase = r * seq
            for k in range(nsub):
                v = slab[r, pl.ds(k * _VL, _VL)]
                dst = dbase + k * _VL + lane
                if (k + 1) * _VL <= seq:
                    plsc.store_scatter(idx_flat, [dst], v)
                else:
                    plsc.store_scatter(idx_flat, [dst], v,
                                       mask=lane < (seq - k * _VL))

        def start_gather(c, b):
            pltpu.async_copy(
                table_hbm.at[idx_flat.at[pl.ds(c * _CHUNK, _CHUNK)]],
                rows[b], gsem[b],
            )

        def wait_gather(c, b):
            pltpu.make_async_copy(
                table_hbm.at[idx_flat.at[pl.ds(c * _CHUNK, _CHUNK)]],
                rows[b], gsem[b],
            ).wait()

        def start_wb(c, b):
            pltpu.async_copy(
                rows[b], out_hbm.at[pl.ds(base + c * _CHUNK, _CHUNK)], wsem[b]
            )

        def wait_wb(c, b):
            pltpu.make_async_copy(
                rows[b], out_hbm.at[pl.ds(base + c * _CHUNK, _CHUNK)], wsem[b]
            ).wait()

        for i in range(_NBUF):
            start_gather(i, i)
            if i >= _LAG:
                d = i - _LAG
                wait_gather(d, d % _NBUF)
                start_wb(d, d % _NBUF)

        @pl.loop(1, num_groups)
        def _(k):
            c0 = k * _NBUF
            for i in range(_NBUF):
                c = c0 + i
                wait_wb(c - _NBUF, i)
                start_gather(c, i)
                d = c - _LAG
                bd = (i + _NBUF - _LAG) % _NBUF
                wait_gather(d, bd)
                start_wb(d, bd)

        for d in range(num_chunks - _LAG, num_chunks):
            wait_gather(d, d % _NBUF)
            start_wb(d, d % _NBUF)
        for b in range(_NBUF):
            wait_wb(num_chunks - _NBUF + b, b)

    out = gather_kernel(table, xpad)
    return out.reshape(batch, seq, emb)
